# baseline (speedup 1.0000x reference)
import functools

import jax
import jax.numpy as jnp
from jax import lax
from jax.experimental import pallas as pl
from jax.experimental.pallas import tpu as pltpu
from jax.experimental.pallas import tpu_sc as plsc

_B = 128
_POOL = 1000
_K = 5
_EPL = 8
_D = 768
_NL_Q = 5
_GPL = 6
_NC = 2
_NS = 16
_NW = _NC * _NS
_NL = 3
_SUBROWS = _B * _K * _EPL
_RPW = _SUBROWS // _NW
_CHUNK = 40
_NCHUNK = _RPW // _CHUNK
_TOT_CHUNKS = _NL * _NCHUNK
_NBUF = 3


def _score_topk_body(xq, k2, k3, k4, g0, g1, o_ref, o0_ref, o1_ref):
    o0_ref[...] = jnp.broadcast_to(g0[...][None], (_B, _GPL, _D))
    o1_ref[...] = jnp.broadcast_to(g1[...][None], (_B, _GPL, _D))
    lane = lax.broadcasted_iota(jnp.int32, (_B, _K * _EPL), 1)
    colid = lax.broadcasted_iota(jnp.int32, (_B, _POOL), 1)
    for i, k_ref in enumerate((k2, k3, k4)):
        kmat = k_ref[...]
        kn = jnp.maximum(jnp.sqrt(jnp.sum(kmat * kmat, axis=1, keepdims=True)),
                         1e-12)
        nk = kmat / kn
        x = xq[:, (2 + i) * _D:(3 + i) * _D]
        qn = jnp.maximum(jnp.sqrt(jnp.sum(x * x, axis=1, keepdims=True)),
                         1e-12)
        q = x / qn
        s = lax.dot_general(q, nk, (((1,), (1,)), ((), ())),
                            preferred_element_type=jnp.float32)
        acc = jnp.zeros((_B, _K * _EPL), jnp.int32)
        for t in range(_K):
            m = jnp.max(s, axis=1, keepdims=True)
            idx = jnp.min(jnp.where(s == m, colid, jnp.int32(2**30)),
                          axis=1, keepdims=True)
            acc = jnp.where(lane // _EPL == t, idx * _EPL + lane % _EPL, acc)
            s = jnp.where(colid == idx, -jnp.inf, s)
        o_ref[i] = acc


def _score_topk(xq2d, k2, k3, k4, g0, g1, interpret=False):
    return pl.pallas_call(
        _score_topk_body,
        out_shape=(
            jax.ShapeDtypeStruct((_NL, _B, _K * _EPL), jnp.int32),
            jax.ShapeDtypeStruct((_B, _GPL, _D), jnp.float32),
            jax.ShapeDtypeStruct((_B, _GPL, _D), jnp.float32),
        ),
        interpret=interpret,
    )(xq2d, k2, k3, k4, g0, g1)


def _make_gather():
    mesh = plsc.VectorSubcoreMesh(core_axis_name="c", subcore_axis_name="s",
                                  num_cores=_NC, num_subcores=_NS)

    @functools.partial(
        pl.kernel,
        mesh=mesh,
        out_type=[jax.ShapeDtypeStruct((_SUBROWS, _D), jnp.float32)] * _NL,
        scratch_types=[
            pltpu.VMEM((_TOT_CHUNKS, _CHUNK), jnp.int32),
        ] + [pltpu.VMEM((_CHUNK, _D), jnp.float32)] * _NBUF
          + [pltpu.SemaphoreType.DMA] * (2 * _NBUF),
    )
    def gather(t2, t3, t4, idx_hbm, o2, o3, o4, idx_v, *bufsem):
        bufs = bufsem[:_NBUF]
        gsems = bufsem[_NBUF:2 * _NBUF]
        ssems = bufsem[2 * _NBUF:]
        wid = lax.axis_index("s") * _NC + lax.axis_index("c")
        for l in range(_NL):
            pltpu.sync_copy(idx_hbm.at[l * _NW + wid],
                            idx_v.at[pl.ds(l * _NCHUNK, _NCHUNK)])
        tabs = (t2, t3, t4)
        outs = (o2, o3, o4)
        base = wid * _RPW

        def start_gather(c):
            l = c // _NCHUNK
            cp = pltpu.make_async_copy(
                tabs[l].at[idx_v.at[c]], bufs[c % _NBUF], gsems[c % _NBUF])
            cp.start()
            return cp

        def start_scatter(c):
            l, cc = divmod(c, _NCHUNK)
            cp = pltpu.make_async_copy(
                bufs[c % _NBUF],
                outs[l].at[pl.ds(base + cc * _CHUNK, _CHUNK)],
                ssems[c % _NBUF])
            cp.start()
            return cp

        gs = [None] * _TOT_CHUNKS
        ss = [None] * _TOT_CHUNKS
        for c in range(_TOT_CHUNKS):
            if c >= _NBUF:
                ss[c - _NBUF].wait()
            gs[c] = start_gather(c)
            if c >= 1:
                gs[c - 1].wait()
                ss[c - 1] = start_scatter(c - 1)
        gs[_TOT_CHUNKS - 1].wait()
        ss[_TOT_CHUNKS - 1] = start_scatter(_TOT_CHUNKS - 1)
        for c in range(_TOT_CHUNKS - _NBUF, _TOT_CHUNKS):
            ss[c].wait()

    return gather


@functools.lru_cache(maxsize=1)
def _gather_cached():
    return _make_gather()


def kernel(x_query, vis_mark, g_p_0, g_p_1, e_p_2, e_k_2, e_p_3, e_k_3,
           e_p_4, e_k_4):
    xq2d = x_query.reshape(_B, _NL_Q * _D)
    sub, out0, out1 = _score_topk(xq2d, e_k_2, e_k_3, e_k_4, g_p_0, g_p_1)
    idx = sub.reshape(_NL * _NW, _NCHUNK, _CHUNK)
    t2 = e_p_2.reshape(_POOL * _EPL, _D)
    t3 = e_p_3.reshape(_POOL * _EPL, _D)
    t4 = e_p_4.reshape(_POOL * _EPL, _D)
    o2, o3, o4 = _gather_cached()(t2, t3, t4, idx)
    out2 = o2.reshape(_B, _K, _EPL, _D)
    out3 = o3.reshape(_B, _K, _EPL, _D)
    out4 = o4.reshape(_B, _K, _EPL, _D)
    loss = jnp.zeros((), jnp.float32)
    return (out0, out1, out2, out3, out4, loss)
